# baseline (speedup 1.0000x reference)
import functools

import jax
import jax.numpy as jnp
from jax import lax
from jax.experimental import pallas as pl
from jax.experimental.pallas import tpu as pltpu
from jax.experimental.pallas import tpu_sc as plsc

N = 10000
E = 320000
H = 128

NC = 2
NS = 16
NW = NC * NS
F = H // NW
CH = 6400
NCH = E // CH
L = 16


def _enc_body(x_ref, we_ref, be_ref, wm_ref, bm_ref, h_ref, hm_ref):
    x = x_ref[...]
    h = lax.dot_general(x, we_ref[...], (((1,), (1,)), ((), ())),
                        preferred_element_type=jnp.float32)
    h = jnp.maximum(h + be_ref[...], 0.0)
    h_ref[...] = h
    hm = lax.dot_general(h, wm_ref[...], (((1,), (1,)), ((), ())),
                         preferred_element_type=jnp.float32)
    hm_ref[...] = hm + bm_ref[...]


def _encode(x, W_enc, b_enc, Wm1, b_msg):
    blk = 1000
    grid = N // blk
    return pl.pallas_call(
        _enc_body,
        grid=(grid,),
        in_specs=[
            pl.BlockSpec((blk, H), lambda i: (i, 0)),
            pl.BlockSpec((H, H), lambda i: (0, 0)),
            pl.BlockSpec((1, H), lambda i: (0, 0)),
            pl.BlockSpec((H, H), lambda i: (0, 0)),
            pl.BlockSpec((1, H), lambda i: (0, 0)),
        ],
        out_specs=[
            pl.BlockSpec((blk, H), lambda i: (i, 0)),
            pl.BlockSpec((blk, H), lambda i: (i, 0)),
        ],
        out_shape=[
            jax.ShapeDtypeStruct((N, H), jnp.float32),
            jax.ShapeDtypeStruct((N, H), jnp.float32),
        ],
    )(x, W_enc, b_enc.reshape(1, H), Wm1, b_msg.reshape(1, H))


def _perm16(x, idx):
    return lax.gather(
        x, idx.reshape(L, 1),
        lax.GatherDimensionNumbers(
            offset_dims=(), collapsed_slice_dims=(0,), start_index_map=(0,)),
        (1,),
        mode=lax.GatherScatterMode.PROMISE_IN_BOUNDS)


def _segmin_body(hmfs_hbm, src_hbm, dst_hbm, ea_hbm, wcol_hbm, out_hbm,
                 acc, hmf, db, db2, sb, sb2, ab, ab2, wcolv,
                 sems):
    cid = lax.axis_index("c")
    sid = lax.axis_index("s")
    wid = sid * NC + cid

    pltpu.sync_copy(wcol_hbm, wcolv)
    pltpu.sync_copy(hmfs_hbm.at[pl.ds(pl.multiple_of(wid * N * F, 8), N * F)],
                    hmf)

    wsp = [plsc.load_gather(
        wcolv, [jnp.zeros((L,), jnp.int32) + (wid * F + t)])
        for t in range(F)]

    inf16 = jnp.full((L,), jnp.inf, dtype=jnp.float32)

    def _init_acc(i, c):
        acc[pl.ds(i * L, L)] = inf16
        return c

    lax.fori_loop(0, N * F // L, _init_acc, 0)

    iota = lax.iota(jnp.int32, L)
    sidx = [jnp.maximum(iota - s, 0) for s in (1, 2, 4, 8)]
    smask = [iota >= s for s in (1, 2, 4, 8)]
    nidx = jnp.minimum(iota + 1, L - 1)
    last = iota == (L - 1)

    dbs = (db, db2)
    sbs = (sb, sb2)
    abs_ = (ab, ab2)

    for b in range(2):
        ebase = b * CH
        pltpu.async_copy(dst_hbm.at[pl.ds(ebase, CH)], dbs[b], sems.at[3 * b])
        pltpu.async_copy(src_hbm.at[pl.ds(ebase, CH)], sbs[b],
                         sems.at[3 * b + 1])
        pltpu.async_copy(ea_hbm.at[pl.ds(ebase, CH)], abs_[b],
                         sems.at[3 * b + 2])

    def _vreg(k, c, dbuf, sbuf, abuf):
        dv = dbuf[pl.ds(k * L, L)]
        sv = sbuf[pl.ds(k * L, L)]
        av = abuf[pl.ds(k * L, L)]
        dk, perm = plsc.sort_key_val(dv, iota)
        svp = _perm16(sv, perm)
        avp = _perm16(av, perm)
        sv4 = svp * F
        dk4 = dk * F
        eq = [smask[i] & (dk == _perm16(dk, sidx[i])) for i in range(4)]
        tail = last | (dk != _perm16(dk, nidx))
        for t in range(F):
            g = plsc.load_gather(hmf, [sv4 + t])
            msg = g + avp * wsp[t]
            for i in range(4):
                sh = _perm16(msg, sidx[i])
                msg = jnp.where(eq[i], jnp.minimum(msg, sh), msg)
            cur = plsc.load_gather(acc, [dk4 + t])
            plsc.store_scatter(acc, [dk4 + t],
                               jnp.minimum(cur, msg), mask=tail)
        return c

    def _pair(g2, carry):
        for b in range(2):
            ci = g2 * 2 + b
            dbuf, sbuf, abuf = dbs[b], sbs[b], abs_[b]
            pltpu.make_async_copy(dst_hbm.at[pl.ds(0, CH)], dbuf,
                                  sems.at[3 * b]).wait()
            pltpu.make_async_copy(src_hbm.at[pl.ds(0, CH)], sbuf,
                                  sems.at[3 * b + 1]).wait()
            pltpu.make_async_copy(ea_hbm.at[pl.ds(0, CH)], abuf,
                                  sems.at[3 * b + 2]).wait()

            body = functools.partial(_vreg, dbuf=dbuf, sbuf=sbuf, abuf=abuf)
            lax.fori_loop(0, CH // L, body, 0)

            @pl.when(ci + 2 < NCH)
            def _pf():
                nxt = pl.multiple_of((ci + 2) * CH, 8)
                pltpu.async_copy(dst_hbm.at[pl.ds(nxt, CH)], dbuf,
                                 sems.at[3 * b])
                pltpu.async_copy(src_hbm.at[pl.ds(nxt, CH)], sbuf,
                                 sems.at[3 * b + 1])
                pltpu.async_copy(ea_hbm.at[pl.ds(nxt, CH)], abuf,
                                 sems.at[3 * b + 2])
        return carry

    lax.fori_loop(0, NCH // 2, _pair, 0)

    pltpu.sync_copy(acc,
                    out_hbm.at[pl.ds(pl.multiple_of(wid * N * F, 8), N * F)])


def _segment_min(hmfs, src, dst, ea, wcol):
    mesh = plsc.VectorSubcoreMesh(core_axis_name="c", subcore_axis_name="s",
                                  num_cores=NC, num_subcores=NS)
    f = pl.kernel(
        _segmin_body,
        out_type=jax.ShapeDtypeStruct((NW * N * F,), jnp.float32),
        mesh=mesh,
        compiler_params=pltpu.CompilerParams(needs_layout_passes=False),
        scratch_types=[
            pltpu.VMEM((N * F,), jnp.float32),
            pltpu.VMEM((N * F,), jnp.float32),
            pltpu.VMEM((CH,), jnp.int32),
            pltpu.VMEM((CH,), jnp.int32),
            pltpu.VMEM((CH,), jnp.int32),
            pltpu.VMEM((CH,), jnp.int32),
            pltpu.VMEM((CH,), jnp.float32),
            pltpu.VMEM((CH,), jnp.float32),
            pltpu.VMEM((H,), jnp.float32),
            pltpu.SemaphoreType.DMA((6,)),
        ],
    )
    out = f(hmfs, src, dst, ea, wcol)
    return out.reshape(NW, N, F).transpose(1, 0, 2).reshape(N, H)


def _dec_body(h_ref, ag_ref, wu1_ref, wu2_ref, bu_ref, wd_ref, bd_ref, o_ref):
    upd = lax.dot_general(h_ref[...], wu1_ref[...], (((1,), (1,)), ((), ())),
                          preferred_element_type=jnp.float32)
    upd = upd + lax.dot_general(ag_ref[...], wu2_ref[...],
                                (((1,), (1,)), ((), ())),
                                preferred_element_type=jnp.float32)
    upd = upd + bu_ref[...]
    o = jnp.sum(upd * wd_ref[...], axis=1, keepdims=True)
    o_ref[...] = jax.nn.sigmoid(o + bd_ref[0, 0])


def _decode(h, aggr, Wu1, Wu2, b_upd, W_dec, b_dec):
    blk = 1000
    grid = N // blk
    return pl.pallas_call(
        _dec_body,
        grid=(grid,),
        in_specs=[
            pl.BlockSpec((blk, H), lambda i: (i, 0)),
            pl.BlockSpec((blk, H), lambda i: (i, 0)),
            pl.BlockSpec((H, H), lambda i: (0, 0)),
            pl.BlockSpec((H, H), lambda i: (0, 0)),
            pl.BlockSpec((1, H), lambda i: (0, 0)),
            pl.BlockSpec((1, H), lambda i: (0, 0)),
            pl.BlockSpec((1, 1), lambda i: (0, 0)),
        ],
        out_specs=pl.BlockSpec((blk, 1), lambda i: (i, 0)),
        out_shape=jax.ShapeDtypeStruct((N, 1), jnp.float32),
    )(h, aggr, Wu1, Wu2, b_upd.reshape(1, H), W_dec, b_dec.reshape(1, 1))


def kernel(x, edge_index, edge_attr, W_enc, b_enc, W_msg, b_msg,
           W_upd, b_upd, W_dec, b_dec):
    src = edge_index[0]
    dst = edge_index[1]
    Wm1 = W_msg[:, :H]
    wcol = W_msg[:, H]
    Wu1 = W_upd[:, :H]
    Wu2 = W_upd[:, H:]

    h, hm = _encode(x, W_enc, b_enc, Wm1, b_msg)
    hmfs = hm.reshape(N, NW, F).transpose(1, 0, 2).reshape(NW * N * F)
    aggr = _segment_min(hmfs, src, dst, edge_attr, wcol)
    return _decode(h, aggr, Wu1, Wu2, b_upd, W_dec, b_dec)


# speedup vs baseline: 1.2539x; 1.2539x over previous
import functools

import jax
import jax.numpy as jnp
from jax import lax
from jax.experimental import pallas as pl
from jax.experimental.pallas import tpu as pltpu
from jax.experimental.pallas import tpu_sc as plsc

N = 10000
E = 320000
H = 128

NC = 2
NS = 16
NW = NC * NS
F = H // NW
CH = 6400
NCH = E // CH
L = 16


def _enc_body(x_ref, we_ref, be_ref, wm_ref, bm_ref, h_ref, hm_ref):
    x = x_ref[...]
    h = lax.dot_general(x, we_ref[...], (((1,), (1,)), ((), ())),
                        preferred_element_type=jnp.float32)
    h = jnp.maximum(h + be_ref[...], 0.0)
    h_ref[...] = h
    hm = lax.dot_general(h, wm_ref[...], (((1,), (1,)), ((), ())),
                         preferred_element_type=jnp.float32)
    hm_ref[...] = hm + bm_ref[...]


def _encode(x, W_enc, b_enc, Wm1, b_msg):
    blk = 1000
    grid = N // blk
    return pl.pallas_call(
        _enc_body,
        grid=(grid,),
        in_specs=[
            pl.BlockSpec((blk, H), lambda i: (i, 0)),
            pl.BlockSpec((H, H), lambda i: (0, 0)),
            pl.BlockSpec((1, H), lambda i: (0, 0)),
            pl.BlockSpec((H, H), lambda i: (0, 0)),
            pl.BlockSpec((1, H), lambda i: (0, 0)),
        ],
        out_specs=[
            pl.BlockSpec((blk, H), lambda i: (i, 0)),
            pl.BlockSpec((blk, H), lambda i: (i, 0)),
        ],
        out_shape=[
            jax.ShapeDtypeStruct((N, H), jnp.float32),
            jax.ShapeDtypeStruct((N, H), jnp.float32),
        ],
    )(x, W_enc, b_enc.reshape(1, H), Wm1, b_msg.reshape(1, H))


def _perm16(x, idx):
    return lax.gather(
        x, idx.reshape(L, 1),
        lax.GatherDimensionNumbers(
            offset_dims=(), collapsed_slice_dims=(0,), start_index_map=(0,)),
        (1,),
        mode=lax.GatherScatterMode.PROMISE_IN_BOUNDS)


def _segmin_body(hmfs_hbm, src_hbm, dst_hbm, ea_hbm, wcol_hbm, out_hbm,
                 acc, hmf, db, db2, sb, sb2, ab, ab2, wcolv,
                 sems):
    cid = lax.axis_index("c")
    sid = lax.axis_index("s")
    wid = sid * NC + cid

    pltpu.sync_copy(wcol_hbm, wcolv)
    pltpu.sync_copy(hmfs_hbm.at[pl.ds(pl.multiple_of(wid * N * F, 8), N * F)],
                    hmf)

    wsp = [plsc.load_gather(
        wcolv, [jnp.zeros((L,), jnp.int32) + (wid * F + t)])
        for t in range(F)]

    inf16 = jnp.full((L,), jnp.inf, dtype=jnp.float32)

    def _init_acc(i, c):
        acc[pl.ds(i * L, L)] = inf16
        return c

    lax.fori_loop(0, N * F // L, _init_acc, 0)

    iota = lax.iota(jnp.int32, L)
    sidx = [jnp.maximum(iota - s, 0) for s in (1, 2, 4, 8)]
    smask = [iota >= s for s in (1, 2, 4, 8)]
    nidx = jnp.minimum(iota + 1, L - 1)
    last = iota == (L - 1)

    dbs = (db, db2)
    sbs = (sb, sb2)
    abs_ = (ab, ab2)

    for b in range(2):
        ebase = b * CH
        pltpu.async_copy(dst_hbm.at[pl.ds(ebase, CH)], dbs[b], sems.at[3 * b])
        pltpu.async_copy(src_hbm.at[pl.ds(ebase, CH)], sbs[b],
                         sems.at[3 * b + 1])
        pltpu.async_copy(ea_hbm.at[pl.ds(ebase, CH)], abs_[b],
                         sems.at[3 * b + 2])

    def _vreg(k, c, dbuf, sbuf, abuf):
        dv = dbuf[pl.ds(k * L, L)]
        sv = sbuf[pl.ds(k * L, L)]
        av = abuf[pl.ds(k * L, L)]
        dk, perm = plsc.sort_key_val(dv, iota)
        tail = last | (dk != _perm16(dk, nidx))
        ntl = plsc.all_reduce_population_count(tail)
        nodup = ntl[0] == L

        @pl.when(nodup)
        def _fast():
            sv4 = sv * F
            dv4 = dv * F
            for t in range(F):
                g = plsc.load_gather(hmf, [sv4 + t])
                msg = g + av * wsp[t]
                cur = plsc.load_gather(acc, [dv4 + t])
                plsc.store_scatter(acc, [dv4 + t], jnp.minimum(cur, msg))

        @pl.when(jnp.logical_not(nodup))
        def _slow():
            svp = _perm16(sv, perm)
            avp = _perm16(av, perm)
            sv4 = svp * F
            dk4 = dk * F
            eq = [smask[i] & (dk == _perm16(dk, sidx[i])) for i in range(4)]
            for t in range(F):
                g = plsc.load_gather(hmf, [sv4 + t])
                msg = g + avp * wsp[t]
                for i in range(4):
                    sh = _perm16(msg, sidx[i])
                    msg = jnp.where(eq[i], jnp.minimum(msg, sh), msg)
                cur = plsc.load_gather(acc, [dk4 + t])
                plsc.store_scatter(acc, [dk4 + t],
                                   jnp.minimum(cur, msg), mask=tail)
        return c

    def _pair(g2, carry):
        for b in range(2):
            ci = g2 * 2 + b
            dbuf, sbuf, abuf = dbs[b], sbs[b], abs_[b]
            pltpu.make_async_copy(dst_hbm.at[pl.ds(0, CH)], dbuf,
                                  sems.at[3 * b]).wait()
            pltpu.make_async_copy(src_hbm.at[pl.ds(0, CH)], sbuf,
                                  sems.at[3 * b + 1]).wait()
            pltpu.make_async_copy(ea_hbm.at[pl.ds(0, CH)], abuf,
                                  sems.at[3 * b + 2]).wait()

            body = functools.partial(_vreg, dbuf=dbuf, sbuf=sbuf, abuf=abuf)
            lax.fori_loop(0, CH // L, body, 0, unroll=2)

            @pl.when(ci + 2 < NCH)
            def _pf():
                nxt = pl.multiple_of((ci + 2) * CH, 8)
                pltpu.async_copy(dst_hbm.at[pl.ds(nxt, CH)], dbuf,
                                 sems.at[3 * b])
                pltpu.async_copy(src_hbm.at[pl.ds(nxt, CH)], sbuf,
                                 sems.at[3 * b + 1])
                pltpu.async_copy(ea_hbm.at[pl.ds(nxt, CH)], abuf,
                                 sems.at[3 * b + 2])
        return carry

    lax.fori_loop(0, NCH // 2, _pair, 0)

    pltpu.sync_copy(acc,
                    out_hbm.at[pl.ds(pl.multiple_of(wid * N * F, 8), N * F)])


def _segment_min(hmfs, src, dst, ea, wcol):
    mesh = plsc.VectorSubcoreMesh(core_axis_name="c", subcore_axis_name="s",
                                  num_cores=NC, num_subcores=NS)
    f = pl.kernel(
        _segmin_body,
        out_type=jax.ShapeDtypeStruct((NW * N * F,), jnp.float32),
        mesh=mesh,
        compiler_params=pltpu.CompilerParams(needs_layout_passes=False),
        scratch_types=[
            pltpu.VMEM((N * F,), jnp.float32),
            pltpu.VMEM((N * F,), jnp.float32),
            pltpu.VMEM((CH,), jnp.int32),
            pltpu.VMEM((CH,), jnp.int32),
            pltpu.VMEM((CH,), jnp.int32),
            pltpu.VMEM((CH,), jnp.int32),
            pltpu.VMEM((CH,), jnp.float32),
            pltpu.VMEM((CH,), jnp.float32),
            pltpu.VMEM((H,), jnp.float32),
            pltpu.SemaphoreType.DMA((6,)),
        ],
    )
    out = f(hmfs, src, dst, ea, wcol)
    return out.reshape(NW, N, F).transpose(1, 0, 2).reshape(N, H)


def _dec_body(h_ref, ag_ref, wu1_ref, wu2_ref, bu_ref, wd_ref, bd_ref, o_ref):
    upd = lax.dot_general(h_ref[...], wu1_ref[...], (((1,), (1,)), ((), ())),
                          preferred_element_type=jnp.float32)
    upd = upd + lax.dot_general(ag_ref[...], wu2_ref[...],
                                (((1,), (1,)), ((), ())),
                                preferred_element_type=jnp.float32)
    upd = upd + bu_ref[...]
    o = jnp.sum(upd * wd_ref[...], axis=1, keepdims=True)
    o_ref[...] = jax.nn.sigmoid(o + bd_ref[0, 0])


def _decode(h, aggr, Wu1, Wu2, b_upd, W_dec, b_dec):
    blk = 1000
    grid = N // blk
    return pl.pallas_call(
        _dec_body,
        grid=(grid,),
        in_specs=[
            pl.BlockSpec((blk, H), lambda i: (i, 0)),
            pl.BlockSpec((blk, H), lambda i: (i, 0)),
            pl.BlockSpec((H, H), lambda i: (0, 0)),
            pl.BlockSpec((H, H), lambda i: (0, 0)),
            pl.BlockSpec((1, H), lambda i: (0, 0)),
            pl.BlockSpec((1, H), lambda i: (0, 0)),
            pl.BlockSpec((1, 1), lambda i: (0, 0)),
        ],
        out_specs=pl.BlockSpec((blk, 1), lambda i: (i, 0)),
        out_shape=jax.ShapeDtypeStruct((N, 1), jnp.float32),
    )(h, aggr, Wu1, Wu2, b_upd.reshape(1, H), W_dec, b_dec.reshape(1, 1))


def kernel(x, edge_index, edge_attr, W_enc, b_enc, W_msg, b_msg,
           W_upd, b_upd, W_dec, b_dec):
    src = edge_index[0]
    dst = edge_index[1]
    Wm1 = W_msg[:, :H]
    wcol = W_msg[:, H]
    Wu1 = W_upd[:, :H]
    Wu2 = W_upd[:, H:]

    h, hm = _encode(x, W_enc, b_enc, Wm1, b_msg)
    hmfs = hm.reshape(N, NW, F).transpose(1, 0, 2).reshape(NW * N * F)
    aggr = _segment_min(hmfs, src, dst, edge_attr, wcol)
    return _decode(h, aggr, Wu1, Wu2, b_upd, W_dec, b_dec)


# speedup vs baseline: 1.9853x; 1.5833x over previous
import functools

import jax
import jax.numpy as jnp
from jax import lax
from jax.experimental import pallas as pl
from jax.experimental.pallas import tpu as pltpu
from jax.experimental.pallas import tpu_sc as plsc

N = 10000
E = 320000
H = 128

NC = 2
NS = 16
NW = NC * NS
F = H // NW
CH = 6400
NCH = E // CH
L = 16
TM = 4096


def _enc_body(x_ref, we_ref, be_ref, wm_ref, bm_ref, h_ref, hm_ref):
    x = x_ref[...]
    h = lax.dot_general(x, we_ref[...], (((1,), (1,)), ((), ())),
                        preferred_element_type=jnp.float32)
    h = jnp.maximum(h + be_ref[...], 0.0)
    h_ref[...] = h
    hm = lax.dot_general(h, wm_ref[...], (((1,), (1,)), ((), ())),
                         preferred_element_type=jnp.float32)
    hm_ref[...] = hm + bm_ref[...]


def _encode(x, W_enc, b_enc, Wm1, b_msg):
    blk = 1000
    grid = N // blk
    return pl.pallas_call(
        _enc_body,
        grid=(grid,),
        in_specs=[
            pl.BlockSpec((blk, H), lambda i: (i, 0)),
            pl.BlockSpec((H, H), lambda i: (0, 0)),
            pl.BlockSpec((1, H), lambda i: (0, 0)),
            pl.BlockSpec((H, H), lambda i: (0, 0)),
            pl.BlockSpec((1, H), lambda i: (0, 0)),
        ],
        out_specs=[
            pl.BlockSpec((blk, H), lambda i: (i, 0)),
            pl.BlockSpec((blk, H), lambda i: (i, 0)),
        ],
        out_shape=[
            jax.ShapeDtypeStruct((N, H), jnp.float32),
            jax.ShapeDtypeStruct((N, H), jnp.float32),
        ],
    )(x, W_enc, b_enc.reshape(1, H), Wm1, b_msg.reshape(1, H))


def _perm16(x, idx):
    return lax.gather(
        x, idx.reshape(L, 1),
        lax.GatherDimensionNumbers(
            offset_dims=(), collapsed_slice_dims=(0,), start_index_map=(0,)),
        (1,),
        mode=lax.GatherScatterMode.PROMISE_IN_BOUNDS)


def _segmin_body(hmfs_hbm, src_hbm, dst_hbm, ea_hbm, wcol_hbm, out_hbm,
                 acc0, acc1, acc2, acc3, hmf0, hmf1, hmf2, hmf3,
                 db, db2, sb, sb2, ab, ab2, tmp0, tmp1, wcolv,
                 sems):
    cid = lax.axis_index("c")
    sid = lax.axis_index("s")
    wid = sid * NC + cid
    accs = (acc0, acc1, acc2, acc3)
    hmfs_ = (hmf0, hmf1, hmf2, hmf3)
    tmps = (tmp0, tmp1)

    pltpu.sync_copy(wcol_hbm, wcolv)
    for t in range(F):
        pltpu.sync_copy(
            hmfs_hbm.at[pl.ds(pl.multiple_of((wid * F + t) * N, 8), N)],
            hmfs_[t])

    wsp = [plsc.load_gather(
        wcolv, [jnp.zeros((L,), jnp.int32) + (wid * F + t)])
        for t in range(F)]

    inf16 = jnp.full((L,), jnp.inf, dtype=jnp.float32)

    def _init_acc(i, c):
        for t in range(F):
            accs[t][pl.ds(i * L, L)] = inf16
        return c

    lax.fori_loop(0, N // L, _init_acc, 0)

    iota = lax.iota(jnp.int32, L)
    sidx = [jnp.maximum(iota - s, 0) for s in (1, 2, 4, 8)]
    smask = [iota >= s for s in (1, 2, 4, 8)]
    nidx = jnp.minimum(iota + 1, L - 1)
    last = iota == (L - 1)

    dbs = (db, db2)
    sbs = (sb, sb2)
    abs_ = (ab, ab2)

    for b in range(2):
        ebase = b * CH
        pltpu.async_copy(dst_hbm.at[pl.ds(ebase, CH)], dbs[b], sems.at[3 * b])
        pltpu.async_copy(src_hbm.at[pl.ds(ebase, CH)], sbs[b],
                         sems.at[3 * b + 1])
        pltpu.async_copy(ea_hbm.at[pl.ds(ebase, CH)], abs_[b],
                         sems.at[3 * b + 2])

    def _one(k, dbuf, sbuf, abuf, tmp):
        dv = dbuf[pl.ds(k * L, L)]
        sv = sbuf[pl.ds(k * L, L)]
        av = abuf[pl.ds(k * L, L)]
        hh = dv & (TM - 1)
        plsc.store_scatter(tmp, [hh], iota)
        rd = plsc.load_gather(tmp, [hh])
        pop = plsc.all_reduce_population_count(rd == iota)
        nodup = pop[0] == L

        @pl.when(nodup)
        def _fast():
            for t in range(F):
                g = plsc.load_gather(hmfs_[t], [sv])
                msg = g + av * wsp[t]
                cur = plsc.load_gather(accs[t], [dv])
                plsc.store_scatter(accs[t], [dv], jnp.minimum(cur, msg))

        @pl.when(jnp.logical_not(nodup))
        def _slow():
            dk, perm = plsc.sort_key_val(dv, iota)
            svp = _perm16(sv, perm)
            avp = _perm16(av, perm)
            tail = last | (dk != _perm16(dk, nidx))
            eq = [smask[i] & (dk == _perm16(dk, sidx[i])) for i in range(4)]
            for t in range(F):
                g = plsc.load_gather(hmfs_[t], [svp])
                msg = g + avp * wsp[t]
                for i in range(4):
                    sh = _perm16(msg, sidx[i])
                    msg = jnp.where(eq[i], jnp.minimum(msg, sh), msg)
                cur = plsc.load_gather(accs[t], [dk])
                plsc.store_scatter(accs[t], [dk],
                                   jnp.minimum(cur, msg), mask=tail)

    def _vreg2(k2, c, dbuf, sbuf, abuf):
        for u in range(2):
            _one(k2 * 2 + u, dbuf, sbuf, abuf, tmps[u])
        return c

    def _pair(g2, carry):
        for b in range(2):
            ci = g2 * 2 + b
            dbuf, sbuf, abuf = dbs[b], sbs[b], abs_[b]
            pltpu.make_async_copy(dst_hbm.at[pl.ds(0, CH)], dbuf,
                                  sems.at[3 * b]).wait()
            pltpu.make_async_copy(src_hbm.at[pl.ds(0, CH)], sbuf,
                                  sems.at[3 * b + 1]).wait()
            pltpu.make_async_copy(ea_hbm.at[pl.ds(0, CH)], abuf,
                                  sems.at[3 * b + 2]).wait()

            body = functools.partial(_vreg2, dbuf=dbuf, sbuf=sbuf, abuf=abuf)
            lax.fori_loop(0, CH // (2 * L), body, 0)

            @pl.when(ci + 2 < NCH)
            def _pf():
                nxt = pl.multiple_of((ci + 2) * CH, 8)
                pltpu.async_copy(dst_hbm.at[pl.ds(nxt, CH)], dbuf,
                                 sems.at[3 * b])
                pltpu.async_copy(src_hbm.at[pl.ds(nxt, CH)], sbuf,
                                 sems.at[3 * b + 1])
                pltpu.async_copy(ea_hbm.at[pl.ds(nxt, CH)], abuf,
                                 sems.at[3 * b + 2])
        return carry

    lax.fori_loop(0, NCH // 2, _pair, 0)

    for t in range(F):
        pltpu.sync_copy(
            accs[t],
            out_hbm.at[pl.ds(pl.multiple_of((wid * F + t) * N, 8), N)])


def _segment_min(hmfs, src, dst, ea, wcol):
    mesh = plsc.VectorSubcoreMesh(core_axis_name="c", subcore_axis_name="s",
                                  num_cores=NC, num_subcores=NS)
    f = pl.kernel(
        _segmin_body,
        out_type=jax.ShapeDtypeStruct((NW * N * F,), jnp.float32),
        mesh=mesh,
        compiler_params=pltpu.CompilerParams(needs_layout_passes=False),
        scratch_types=(
            [pltpu.VMEM((N,), jnp.float32)] * 4 +
            [pltpu.VMEM((N,), jnp.float32)] * 4 +
            [
                pltpu.VMEM((CH,), jnp.int32),
                pltpu.VMEM((CH,), jnp.int32),
                pltpu.VMEM((CH,), jnp.int32),
                pltpu.VMEM((CH,), jnp.int32),
                pltpu.VMEM((CH,), jnp.float32),
                pltpu.VMEM((CH,), jnp.float32),
                pltpu.VMEM((TM,), jnp.int32),
                pltpu.VMEM((TM,), jnp.int32),
                pltpu.VMEM((H,), jnp.float32),
                pltpu.SemaphoreType.DMA((6,)),
            ]),
    )
    out = f(hmfs, src, dst, ea, wcol)
    return out.reshape(H, N).T


def _dec_body(h_ref, ag_ref, wu1_ref, wu2_ref, bu_ref, wd_ref, bd_ref, o_ref):
    upd = lax.dot_general(h_ref[...], wu1_ref[...], (((1,), (1,)), ((), ())),
                          preferred_element_type=jnp.float32)
    upd = upd + lax.dot_general(ag_ref[...], wu2_ref[...],
                                (((1,), (1,)), ((), ())),
                                preferred_element_type=jnp.float32)
    upd = upd + bu_ref[...]
    o = jnp.sum(upd * wd_ref[...], axis=1, keepdims=True)
    o_ref[...] = jax.nn.sigmoid(o + bd_ref[0, 0])


def _decode(h, aggr, Wu1, Wu2, b_upd, W_dec, b_dec):
    blk = 1000
    grid = N // blk
    return pl.pallas_call(
        _dec_body,
        grid=(grid,),
        in_specs=[
            pl.BlockSpec((blk, H), lambda i: (i, 0)),
            pl.BlockSpec((blk, H), lambda i: (i, 0)),
            pl.BlockSpec((H, H), lambda i: (0, 0)),
            pl.BlockSpec((H, H), lambda i: (0, 0)),
            pl.BlockSpec((1, H), lambda i: (0, 0)),
            pl.BlockSpec((1, H), lambda i: (0, 0)),
            pl.BlockSpec((1, 1), lambda i: (0, 0)),
        ],
        out_specs=pl.BlockSpec((blk, 1), lambda i: (i, 0)),
        out_shape=jax.ShapeDtypeStruct((N, 1), jnp.float32),
    )(h, aggr, Wu1, Wu2, b_upd.reshape(1, H), W_dec, b_dec.reshape(1, 1))


def kernel(x, edge_index, edge_attr, W_enc, b_enc, W_msg, b_msg,
           W_upd, b_upd, W_dec, b_dec):
    src = edge_index[0]
    dst = edge_index[1]
    Wm1 = W_msg[:, :H]
    wcol = W_msg[:, H]
    Wu1 = W_upd[:, :H]
    Wu2 = W_upd[:, H:]

    h, hm = _encode(x, W_enc, b_enc, Wm1, b_msg)
    hmfs = hm.T.reshape(H * N)
    aggr = _segment_min(hmfs, src, dst, edge_attr, wcol)
    return _decode(h, aggr, Wu1, Wu2, b_upd, W_dec, b_dec)


# speedup vs baseline: 2.8403x; 1.4306x over previous
import functools

import jax
import jax.numpy as jnp
from jax import lax
from jax.experimental import pallas as pl
from jax.experimental.pallas import tpu as pltpu
from jax.experimental.pallas import tpu_sc as plsc

N = 10000
E = 320000
H = 128

NC = 2
NS = 16
NW = NC * NS
F = H // NW
CH = 6400
NCH = E // CH
L = 16
TM = 4096


def _enc_body(x_ref, we_ref, be_ref, wm_ref, bm_ref, h_ref, hm_ref):
    x = x_ref[...]
    h = lax.dot_general(x, we_ref[...], (((1,), (1,)), ((), ())),
                        preferred_element_type=jnp.float32)
    h = jnp.maximum(h + be_ref[...], 0.0)
    h_ref[...] = h
    hm = lax.dot_general(h, wm_ref[...], (((1,), (1,)), ((), ())),
                         preferred_element_type=jnp.float32)
    hm_ref[...] = hm + bm_ref[...]


def _encode(x, W_enc, b_enc, Wm1, b_msg):
    blk = 1000
    grid = N // blk
    return pl.pallas_call(
        _enc_body,
        grid=(grid,),
        in_specs=[
            pl.BlockSpec((blk, H), lambda i: (i, 0)),
            pl.BlockSpec((H, H), lambda i: (0, 0)),
            pl.BlockSpec((1, H), lambda i: (0, 0)),
            pl.BlockSpec((H, H), lambda i: (0, 0)),
            pl.BlockSpec((1, H), lambda i: (0, 0)),
        ],
        out_specs=[
            pl.BlockSpec((blk, H), lambda i: (i, 0)),
            pl.BlockSpec((blk, H), lambda i: (i, 0)),
        ],
        out_shape=[
            jax.ShapeDtypeStruct((N, H), jnp.float32),
            jax.ShapeDtypeStruct((N, H), jnp.float32),
        ],
    )(x, W_enc, b_enc.reshape(1, H), Wm1, b_msg.reshape(1, H))


def _perm16(x, idx):
    return lax.gather(
        x, idx.reshape(L, 1),
        lax.GatherDimensionNumbers(
            offset_dims=(), collapsed_slice_dims=(0,), start_index_map=(0,)),
        (1,),
        mode=lax.GatherScatterMode.PROMISE_IN_BOUNDS)


def _segmin_body(hmfs_hbm, src_hbm, dst_hbm, ea_hbm, wcol_hbm, out_hbm,
                 acc0, acc1, acc2, acc3, hmf0, hmf1, hmf2, hmf3,
                 db, db2, sb, sb2, ab, ab2, tmp0, tmp1, wcolv,
                 sems):
    cid = lax.axis_index("c")
    sid = lax.axis_index("s")
    wid = sid * NC + cid
    accs = (acc0, acc1, acc2, acc3)
    hmfs_ = (hmf0, hmf1, hmf2, hmf3)
    tmps = (tmp0, tmp1)

    pltpu.sync_copy(wcol_hbm, wcolv)
    for t in range(F):
        pltpu.sync_copy(
            hmfs_hbm.at[pl.ds(pl.multiple_of((wid * F + t) * N, 8), N)],
            hmfs_[t])

    wsp = [plsc.load_gather(
        wcolv, [jnp.zeros((L,), jnp.int32) + (wid * F + t)])
        for t in range(F)]

    inf16 = jnp.full((L,), jnp.inf, dtype=jnp.float32)

    def _init_acc(i, c):
        for t in range(F):
            accs[t][pl.ds(i * L, L)] = inf16
        return c

    lax.fori_loop(0, N // L, _init_acc, 0)

    iota = lax.iota(jnp.int32, L)
    sidx = [jnp.maximum(iota - s, 0) for s in (1, 2, 4, 8)]
    smask = [iota >= s for s in (1, 2, 4, 8)]
    nidx = jnp.minimum(iota + 1, L - 1)
    last = iota == (L - 1)

    dbs = (db, db2)
    sbs = (sb, sb2)
    abs_ = (ab, ab2)

    for b in range(2):
        ebase = b * CH
        pltpu.async_copy(dst_hbm.at[pl.ds(ebase, CH)], dbs[b], sems.at[3 * b])
        pltpu.async_copy(src_hbm.at[pl.ds(ebase, CH)], sbs[b],
                         sems.at[3 * b + 1])
        pltpu.async_copy(ea_hbm.at[pl.ds(ebase, CH)], abs_[b],
                         sems.at[3 * b + 2])

    def _one(k, dbuf, sbuf, abuf, tmp):
        dv = dbuf[pl.ds(k * L, L)]
        sv = sbuf[pl.ds(k * L, L)]
        av = abuf[pl.ds(k * L, L)]
        hh = dv & (TM - 1)
        plsc.store_scatter(tmp, [hh], iota)
        rd = plsc.load_gather(tmp, [hh])
        pop = plsc.all_reduce_population_count(rd == iota)

        for t in range(F):
            g = plsc.load_gather(hmfs_[t], [sv])
            msg = g + av * wsp[t]
            cur = plsc.load_gather(accs[t], [dv])
            plsc.store_scatter(accs[t], [dv], jnp.minimum(cur, msg))

        @pl.when(pop[0] < L)
        def _slow():
            dk, perm = plsc.sort_key_val(dv, iota)
            svp = _perm16(sv, perm)
            avp = _perm16(av, perm)
            tail = last | (dk != _perm16(dk, nidx))
            eq = [smask[i] & (dk == _perm16(dk, sidx[i])) for i in range(4)]
            for t in range(F):
                g = plsc.load_gather(hmfs_[t], [svp])
                msg = g + avp * wsp[t]
                for i in range(4):
                    sh = _perm16(msg, sidx[i])
                    msg = jnp.where(eq[i], jnp.minimum(msg, sh), msg)
                cur = plsc.load_gather(accs[t], [dk])
                plsc.store_scatter(accs[t], [dk],
                                   jnp.minimum(cur, msg), mask=tail)

    def _vreg2(k2, c, dbuf, sbuf, abuf):
        for u in range(2):
            _one(k2 * 2 + u, dbuf, sbuf, abuf, tmps[u])
        return c

    def _pair(g2, carry):
        for b in range(2):
            ci = g2 * 2 + b
            dbuf, sbuf, abuf = dbs[b], sbs[b], abs_[b]
            pltpu.make_async_copy(dst_hbm.at[pl.ds(0, CH)], dbuf,
                                  sems.at[3 * b]).wait()
            pltpu.make_async_copy(src_hbm.at[pl.ds(0, CH)], sbuf,
                                  sems.at[3 * b + 1]).wait()
            pltpu.make_async_copy(ea_hbm.at[pl.ds(0, CH)], abuf,
                                  sems.at[3 * b + 2]).wait()

            body = functools.partial(_vreg2, dbuf=dbuf, sbuf=sbuf, abuf=abuf)
            lax.fori_loop(0, CH // (2 * L), body, 0)

            @pl.when(ci + 2 < NCH)
            def _pf():
                nxt = pl.multiple_of((ci + 2) * CH, 8)
                pltpu.async_copy(dst_hbm.at[pl.ds(nxt, CH)], dbuf,
                                 sems.at[3 * b])
                pltpu.async_copy(src_hbm.at[pl.ds(nxt, CH)], sbuf,
                                 sems.at[3 * b + 1])
                pltpu.async_copy(ea_hbm.at[pl.ds(nxt, CH)], abuf,
                                 sems.at[3 * b + 2])
        return carry

    lax.fori_loop(0, NCH // 2, _pair, 0)

    for t in range(F):
        pltpu.sync_copy(
            accs[t],
            out_hbm.at[pl.ds(pl.multiple_of((wid * F + t) * N, 8), N)])


def _segment_min(hmfs, src, dst, ea, wcol):
    mesh = plsc.VectorSubcoreMesh(core_axis_name="c", subcore_axis_name="s",
                                  num_cores=NC, num_subcores=NS)
    f = pl.kernel(
        _segmin_body,
        out_type=jax.ShapeDtypeStruct((NW * N * F,), jnp.float32),
        mesh=mesh,
        compiler_params=pltpu.CompilerParams(needs_layout_passes=False),
        scratch_types=(
            [pltpu.VMEM((N,), jnp.float32)] * 4 +
            [pltpu.VMEM((N,), jnp.float32)] * 4 +
            [
                pltpu.VMEM((CH,), jnp.int32),
                pltpu.VMEM((CH,), jnp.int32),
                pltpu.VMEM((CH,), jnp.int32),
                pltpu.VMEM((CH,), jnp.int32),
                pltpu.VMEM((CH,), jnp.float32),
                pltpu.VMEM((CH,), jnp.float32),
                pltpu.VMEM((TM,), jnp.int32),
                pltpu.VMEM((TM,), jnp.int32),
                pltpu.VMEM((H,), jnp.float32),
                pltpu.SemaphoreType.DMA((6,)),
            ]),
    )
    out = f(hmfs, src, dst, ea, wcol)
    return out.reshape(H, N).T


def _dec_body(h_ref, ag_ref, wu1_ref, wu2_ref, bu_ref, wd_ref, bd_ref, o_ref):
    upd = lax.dot_general(h_ref[...], wu1_ref[...], (((1,), (1,)), ((), ())),
                          preferred_element_type=jnp.float32)
    upd = upd + lax.dot_general(ag_ref[...], wu2_ref[...],
                                (((1,), (1,)), ((), ())),
                                preferred_element_type=jnp.float32)
    upd = upd + bu_ref[...]
    o = jnp.sum(upd * wd_ref[...], axis=1, keepdims=True)
    o_ref[...] = jax.nn.sigmoid(o + bd_ref[0, 0])


def _decode(h, aggr, Wu1, Wu2, b_upd, W_dec, b_dec):
    blk = 1000
    grid = N // blk
    return pl.pallas_call(
        _dec_body,
        grid=(grid,),
        in_specs=[
            pl.BlockSpec((blk, H), lambda i: (i, 0)),
            pl.BlockSpec((blk, H), lambda i: (i, 0)),
            pl.BlockSpec((H, H), lambda i: (0, 0)),
            pl.BlockSpec((H, H), lambda i: (0, 0)),
            pl.BlockSpec((1, H), lambda i: (0, 0)),
            pl.BlockSpec((1, H), lambda i: (0, 0)),
            pl.BlockSpec((1, 1), lambda i: (0, 0)),
        ],
        out_specs=pl.BlockSpec((blk, 1), lambda i: (i, 0)),
        out_shape=jax.ShapeDtypeStruct((N, 1), jnp.float32),
    )(h, aggr, Wu1, Wu2, b_upd.reshape(1, H), W_dec, b_dec.reshape(1, 1))


def kernel(x, edge_index, edge_attr, W_enc, b_enc, W_msg, b_msg,
           W_upd, b_upd, W_dec, b_dec):
    src = edge_index[0]
    dst = edge_index[1]
    Wm1 = W_msg[:, :H]
    wcol = W_msg[:, H]
    Wu1 = W_upd[:, :H]
    Wu2 = W_upd[:, H:]

    h, hm = _encode(x, W_enc, b_enc, Wm1, b_msg)
    hmfs = hm.T.reshape(H * N)
    aggr = _segment_min(hmfs, src, dst, edge_attr, wcol)
    return _decode(h, aggr, Wu1, Wu2, b_upd, W_dec, b_dec)


# speedup vs baseline: 3.2131x; 1.1313x over previous
import functools

import jax
import jax.numpy as jnp
from jax import lax
from jax.experimental import pallas as pl
from jax.experimental.pallas import tpu as pltpu
from jax.experimental.pallas import tpu_sc as plsc

N = 10000
E = 320000
H = 128

NC = 2
NS = 16
NW = NC * NS
F = H // NW
CH = 6400
NCH = E // CH
L = 16
TM = 4096


def _enc_body(x_ref, we_ref, be_ref, wm_ref, bm_ref, h_ref, hm_ref):
    x = x_ref[...]
    h = lax.dot_general(x, we_ref[...], (((1,), (1,)), ((), ())),
                        preferred_element_type=jnp.float32)
    h = jnp.maximum(h + be_ref[...], 0.0)
    h_ref[...] = h
    hm = lax.dot_general(h, wm_ref[...], (((1,), (1,)), ((), ())),
                         preferred_element_type=jnp.float32)
    hm_ref[...] = hm + bm_ref[...]


def _encode(x, W_enc, b_enc, Wm1, b_msg):
    blk = 1000
    grid = N // blk
    return pl.pallas_call(
        _enc_body,
        grid=(grid,),
        in_specs=[
            pl.BlockSpec((blk, H), lambda i: (i, 0)),
            pl.BlockSpec((H, H), lambda i: (0, 0)),
            pl.BlockSpec((1, H), lambda i: (0, 0)),
            pl.BlockSpec((H, H), lambda i: (0, 0)),
            pl.BlockSpec((1, H), lambda i: (0, 0)),
        ],
        out_specs=[
            pl.BlockSpec((blk, H), lambda i: (i, 0)),
            pl.BlockSpec((blk, H), lambda i: (i, 0)),
        ],
        out_shape=[
            jax.ShapeDtypeStruct((N, H), jnp.float32),
            jax.ShapeDtypeStruct((N, H), jnp.float32),
        ],
    )(x, W_enc, b_enc.reshape(1, H), Wm1, b_msg.reshape(1, H))


def _perm16(x, idx):
    return lax.gather(
        x, idx.reshape(L, 1),
        lax.GatherDimensionNumbers(
            offset_dims=(), collapsed_slice_dims=(0,), start_index_map=(0,)),
        (1,),
        mode=lax.GatherScatterMode.PROMISE_IN_BOUNDS)


def _segmin_body(hmfs_hbm, src_hbm, dst_hbm, ea_hbm, wcol_hbm, out_hbm,
                 acc0, acc1, acc2, acc3, hmf0, hmf1, hmf2, hmf3,
                 db, db2, sb, sb2, ab, ab2, wcolv,
                 sems):
    cid = lax.axis_index("c")
    sid = lax.axis_index("s")
    wid = sid * NC + cid
    accs = (acc0, acc1, acc2, acc3)
    hmfs_ = (hmf0, hmf1, hmf2, hmf3)

    pltpu.sync_copy(wcol_hbm, wcolv)
    for t in range(F):
        pltpu.sync_copy(
            hmfs_hbm.at[pl.ds(pl.multiple_of((wid * F + t) * N, 8), N)],
            hmfs_[t])

    wsp = [plsc.load_gather(
        wcolv, [jnp.zeros((L,), jnp.int32) + (wid * F + t)])
        for t in range(F)]

    inf16 = jnp.full((L,), jnp.inf, dtype=jnp.float32)

    def _init_acc(i, c):
        for t in range(F):
            accs[t][pl.ds(i * L, L)] = inf16
        return c

    lax.fori_loop(0, N // L, _init_acc, 0)

    iota = lax.iota(jnp.int32, L)
    sidx = [jnp.maximum(iota - s, 0) for s in (1, 2, 4, 8)]
    smask = [iota >= s for s in (1, 2, 4, 8)]
    nidx = jnp.minimum(iota + 1, L - 1)
    last = iota == (L - 1)

    dbs = (db, db2)
    sbs = (sb, sb2)
    abs_ = (ab, ab2)

    for b in range(2):
        ebase = b * CH
        pltpu.async_copy(dst_hbm.at[pl.ds(ebase, CH)], dbs[b], sems.at[3 * b])
        pltpu.async_copy(src_hbm.at[pl.ds(ebase, CH)], sbs[b],
                         sems.at[3 * b + 1])
        pltpu.async_copy(ea_hbm.at[pl.ds(ebase, CH)], abs_[b],
                         sems.at[3 * b + 2])

    def _one(k, dbuf, sbuf, abuf):
        dv = dbuf[pl.ds(k * L, L)]
        sv = sbuf[pl.ds(k * L, L)]
        av = abuf[pl.ds(k * L, L)]
        _, ulast = plsc.scan_count(dv)
        upop = plsc.all_reduce_population_count(ulast)

        for t in range(F):
            g = plsc.load_gather(hmfs_[t], [sv])
            msg = g + av * wsp[t]
            cur = plsc.load_gather(accs[t], [dv])
            plsc.store_scatter(accs[t], [dv], jnp.minimum(cur, msg))

        @pl.when(upop[0] < L)
        def _slow():
            dk, perm = plsc.sort_key_val(dv, iota)
            svp = _perm16(sv, perm)
            avp = _perm16(av, perm)
            tail = last | (dk != _perm16(dk, nidx))
            eq = [smask[i] & (dk == _perm16(dk, sidx[i])) for i in range(4)]
            for t in range(F):
                g = plsc.load_gather(hmfs_[t], [svp])
                msg = g + avp * wsp[t]
                for i in range(4):
                    sh = _perm16(msg, sidx[i])
                    msg = jnp.where(eq[i], jnp.minimum(msg, sh), msg)
                cur = plsc.load_gather(accs[t], [dk])
                plsc.store_scatter(accs[t], [dk],
                                   jnp.minimum(cur, msg), mask=tail)

    def _vreg4(k4, c, dbuf, sbuf, abuf):
        for u in range(4):
            _one(k4 * 4 + u, dbuf, sbuf, abuf)
        return c

    def _pair(g2, carry):
        for b in range(2):
            ci = g2 * 2 + b
            dbuf, sbuf, abuf = dbs[b], sbs[b], abs_[b]
            pltpu.make_async_copy(dst_hbm.at[pl.ds(0, CH)], dbuf,
                                  sems.at[3 * b]).wait()
            pltpu.make_async_copy(src_hbm.at[pl.ds(0, CH)], sbuf,
                                  sems.at[3 * b + 1]).wait()
            pltpu.make_async_copy(ea_hbm.at[pl.ds(0, CH)], abuf,
                                  sems.at[3 * b + 2]).wait()

            body = functools.partial(_vreg4, dbuf=dbuf, sbuf=sbuf, abuf=abuf)
            lax.fori_loop(0, CH // (4 * L), body, 0)

            @pl.when(ci + 2 < NCH)
            def _pf():
                nxt = pl.multiple_of((ci + 2) * CH, 8)
                pltpu.async_copy(dst_hbm.at[pl.ds(nxt, CH)], dbuf,
                                 sems.at[3 * b])
                pltpu.async_copy(src_hbm.at[pl.ds(nxt, CH)], sbuf,
                                 sems.at[3 * b + 1])
                pltpu.async_copy(ea_hbm.at[pl.ds(nxt, CH)], abuf,
                                 sems.at[3 * b + 2])
        return carry

    lax.fori_loop(0, NCH // 2, _pair, 0)

    for t in range(F):
        pltpu.sync_copy(
            accs[t],
            out_hbm.at[pl.ds(pl.multiple_of((wid * F + t) * N, 8), N)])


def _segment_min(hmfs, src, dst, ea, wcol):
    mesh = plsc.VectorSubcoreMesh(core_axis_name="c", subcore_axis_name="s",
                                  num_cores=NC, num_subcores=NS)
    f = pl.kernel(
        _segmin_body,
        out_type=jax.ShapeDtypeStruct((NW * N * F,), jnp.float32),
        mesh=mesh,
        compiler_params=pltpu.CompilerParams(needs_layout_passes=False),
        scratch_types=(
            [pltpu.VMEM((N,), jnp.float32)] * 4 +
            [pltpu.VMEM((N,), jnp.float32)] * 4 +
            [
                pltpu.VMEM((CH,), jnp.int32),
                pltpu.VMEM((CH,), jnp.int32),
                pltpu.VMEM((CH,), jnp.int32),
                pltpu.VMEM((CH,), jnp.int32),
                pltpu.VMEM((CH,), jnp.float32),
                pltpu.VMEM((CH,), jnp.float32),
                pltpu.VMEM((H,), jnp.float32),
                pltpu.SemaphoreType.DMA((6,)),
            ]),
    )
    out = f(hmfs, src, dst, ea, wcol)
    return out.reshape(H, N).T


def _dec_body(h_ref, ag_ref, wu1_ref, wu2_ref, bu_ref, wd_ref, bd_ref, o_ref):
    upd = lax.dot_general(h_ref[...], wu1_ref[...], (((1,), (1,)), ((), ())),
                          preferred_element_type=jnp.float32)
    upd = upd + lax.dot_general(ag_ref[...], wu2_ref[...],
                                (((1,), (1,)), ((), ())),
                                preferred_element_type=jnp.float32)
    upd = upd + bu_ref[...]
    o = jnp.sum(upd * wd_ref[...], axis=1, keepdims=True)
    o_ref[...] = jax.nn.sigmoid(o + bd_ref[0, 0])


def _decode(h, aggr, Wu1, Wu2, b_upd, W_dec, b_dec):
    blk = 1000
    grid = N // blk
    return pl.pallas_call(
        _dec_body,
        grid=(grid,),
        in_specs=[
            pl.BlockSpec((blk, H), lambda i: (i, 0)),
            pl.BlockSpec((blk, H), lambda i: (i, 0)),
            pl.BlockSpec((H, H), lambda i: (0, 0)),
            pl.BlockSpec((H, H), lambda i: (0, 0)),
            pl.BlockSpec((1, H), lambda i: (0, 0)),
            pl.BlockSpec((1, H), lambda i: (0, 0)),
            pl.BlockSpec((1, 1), lambda i: (0, 0)),
        ],
        out_specs=pl.BlockSpec((blk, 1), lambda i: (i, 0)),
        out_shape=jax.ShapeDtypeStruct((N, 1), jnp.float32),
    )(h, aggr, Wu1, Wu2, b_upd.reshape(1, H), W_dec, b_dec.reshape(1, 1))


def kernel(x, edge_index, edge_attr, W_enc, b_enc, W_msg, b_msg,
           W_upd, b_upd, W_dec, b_dec):
    src = edge_index[0]
    dst = edge_index[1]
    Wm1 = W_msg[:, :H]
    wcol = W_msg[:, H]
    Wu1 = W_upd[:, :H]
    Wu2 = W_upd[:, H:]

    h, hm = _encode(x, W_enc, b_enc, Wm1, b_msg)
    hmfs = hm.T.reshape(H * N)
    aggr = _segment_min(hmfs, src, dst, edge_attr, wcol)
    return _decode(h, aggr, Wu1, Wu2, b_upd, W_dec, b_dec)


# speedup vs baseline: 3.2152x; 1.0007x over previous
import functools

import jax
import jax.numpy as jnp
from jax import lax
from jax.experimental import pallas as pl
from jax.experimental.pallas import tpu as pltpu
from jax.experimental.pallas import tpu_sc as plsc

N = 10000
E = 320000
H = 128

NC = 2
NS = 16
NW = NC * NS
F = H // NW
CH = 6400
NCH = E // CH
L = 16
TM = 4096


def _enc_body(x_ref, we_ref, be_ref, wm_ref, bm_ref, h_ref, hm_ref):
    x = x_ref[...]
    h = lax.dot_general(x, we_ref[...], (((1,), (1,)), ((), ())),
                        preferred_element_type=jnp.float32)
    h = jnp.maximum(h + be_ref[...], 0.0)
    h_ref[...] = h
    hm = lax.dot_general(h, wm_ref[...], (((1,), (1,)), ((), ())),
                         preferred_element_type=jnp.float32)
    hm_ref[...] = hm + bm_ref[...]


def _encode(x, W_enc, b_enc, Wm1, b_msg):
    blk = 1000
    grid = N // blk
    return pl.pallas_call(
        _enc_body,
        grid=(grid,),
        in_specs=[
            pl.BlockSpec((blk, H), lambda i: (i, 0)),
            pl.BlockSpec((H, H), lambda i: (0, 0)),
            pl.BlockSpec((1, H), lambda i: (0, 0)),
            pl.BlockSpec((H, H), lambda i: (0, 0)),
            pl.BlockSpec((1, H), lambda i: (0, 0)),
        ],
        out_specs=[
            pl.BlockSpec((blk, H), lambda i: (i, 0)),
            pl.BlockSpec((blk, H), lambda i: (i, 0)),
        ],
        out_shape=[
            jax.ShapeDtypeStruct((N, H), jnp.float32),
            jax.ShapeDtypeStruct((N, H), jnp.float32),
        ],
    )(x, W_enc, b_enc.reshape(1, H), Wm1, b_msg.reshape(1, H))


def _perm16(x, idx):
    return lax.gather(
        x, idx.reshape(L, 1),
        lax.GatherDimensionNumbers(
            offset_dims=(), collapsed_slice_dims=(0,), start_index_map=(0,)),
        (1,),
        mode=lax.GatherScatterMode.PROMISE_IN_BOUNDS)


def _segmin_body(hmfs_hbm, src_hbm, dst_hbm, ea_hbm, wcol_hbm, out_hbm,
                 acc0, acc1, acc2, acc3, hmf0, hmf1, hmf2, hmf3,
                 db, db2, sb, sb2, ab, ab2, wcolv,
                 sems):
    cid = lax.axis_index("c")
    sid = lax.axis_index("s")
    wid = sid * NC + cid
    accs = (acc0, acc1, acc2, acc3)
    hmfs_ = (hmf0, hmf1, hmf2, hmf3)

    pltpu.sync_copy(wcol_hbm, wcolv)
    for t in range(F):
        pltpu.sync_copy(
            hmfs_hbm.at[pl.ds(pl.multiple_of((wid * F + t) * N, 8), N)],
            hmfs_[t])

    wsp = [plsc.load_gather(
        wcolv, [jnp.zeros((L,), jnp.int32) + (wid * F + t)])
        for t in range(F)]

    inf16 = jnp.full((L,), jnp.inf, dtype=jnp.float32)

    def _init_acc(i, c):
        for t in range(F):
            accs[t][pl.ds(i * L, L)] = inf16
        return c

    lax.fori_loop(0, N // L, _init_acc, 0)

    iota = lax.iota(jnp.int32, L)
    sidx = [jnp.maximum(iota - s, 0) for s in (1, 2, 4, 8)]
    smask = [iota >= s for s in (1, 2, 4, 8)]
    nidx = jnp.minimum(iota + 1, L - 1)
    last = iota == (L - 1)

    dbs = (db, db2)
    sbs = (sb, sb2)
    abs_ = (ab, ab2)

    for b in range(2):
        ebase = b * CH
        pltpu.async_copy(dst_hbm.at[pl.ds(ebase, CH)], dbs[b], sems.at[3 * b])
        pltpu.async_copy(src_hbm.at[pl.ds(ebase, CH)], sbs[b],
                         sems.at[3 * b + 1])
        pltpu.async_copy(ea_hbm.at[pl.ds(ebase, CH)], abs_[b],
                         sems.at[3 * b + 2])

    def _one(k, dbuf, sbuf, abuf):
        dv = dbuf[pl.ds(k * L, L)]
        sv = sbuf[pl.ds(k * L, L)]
        av = abuf[pl.ds(k * L, L)]
        _, ulast = plsc.scan_count(dv)
        upop = plsc.all_reduce_population_count(ulast)

        for t in range(F):
            g = plsc.load_gather(hmfs_[t], [sv])
            msg = g + av * wsp[t]
            cur = plsc.load_gather(accs[t], [dv])
            plsc.store_scatter(accs[t], [dv], jnp.minimum(cur, msg))

        @pl.when(upop[0] < L)
        def _slow():
            dk, perm = plsc.sort_key_val(dv, iota)
            svp = _perm16(sv, perm)
            avp = _perm16(av, perm)
            tail = last | (dk != _perm16(dk, nidx))
            eq = [smask[i] & (dk == _perm16(dk, sidx[i])) for i in range(4)]
            for t in range(F):
                g = plsc.load_gather(hmfs_[t], [svp])
                msg = g + avp * wsp[t]
                for i in range(4):
                    sh = _perm16(msg, sidx[i])
                    msg = jnp.where(eq[i], jnp.minimum(msg, sh), msg)
                cur = plsc.load_gather(accs[t], [dk])
                plsc.store_scatter(accs[t], [dk],
                                   jnp.minimum(cur, msg), mask=tail)

    def _vreg8(k8, c, dbuf, sbuf, abuf):
        for u in range(8):
            _one(k8 * 8 + u, dbuf, sbuf, abuf)
        return c

    def _pair(g2, carry):
        for b in range(2):
            ci = g2 * 2 + b
            dbuf, sbuf, abuf = dbs[b], sbs[b], abs_[b]
            pltpu.make_async_copy(dst_hbm.at[pl.ds(0, CH)], dbuf,
                                  sems.at[3 * b]).wait()
            pltpu.make_async_copy(src_hbm.at[pl.ds(0, CH)], sbuf,
                                  sems.at[3 * b + 1]).wait()
            pltpu.make_async_copy(ea_hbm.at[pl.ds(0, CH)], abuf,
                                  sems.at[3 * b + 2]).wait()

            body = functools.partial(_vreg8, dbuf=dbuf, sbuf=sbuf, abuf=abuf)
            lax.fori_loop(0, CH // (8 * L), body, 0)

            @pl.when(ci + 2 < NCH)
            def _pf():
                nxt = pl.multiple_of((ci + 2) * CH, 8)
                pltpu.async_copy(dst_hbm.at[pl.ds(nxt, CH)], dbuf,
                                 sems.at[3 * b])
                pltpu.async_copy(src_hbm.at[pl.ds(nxt, CH)], sbuf,
                                 sems.at[3 * b + 1])
                pltpu.async_copy(ea_hbm.at[pl.ds(nxt, CH)], abuf,
                                 sems.at[3 * b + 2])
        return carry

    lax.fori_loop(0, NCH // 2, _pair, 0)

    for t in range(F):
        pltpu.sync_copy(
            accs[t],
            out_hbm.at[pl.ds(pl.multiple_of((wid * F + t) * N, 8), N)])


def _segment_min(hmfs, src, dst, ea, wcol):
    mesh = plsc.VectorSubcoreMesh(core_axis_name="c", subcore_axis_name="s",
                                  num_cores=NC, num_subcores=NS)
    f = pl.kernel(
        _segmin_body,
        out_type=jax.ShapeDtypeStruct((NW * N * F,), jnp.float32),
        mesh=mesh,
        compiler_params=pltpu.CompilerParams(needs_layout_passes=False),
        scratch_types=(
            [pltpu.VMEM((N,), jnp.float32)] * 4 +
            [pltpu.VMEM((N,), jnp.float32)] * 4 +
            [
                pltpu.VMEM((CH,), jnp.int32),
                pltpu.VMEM((CH,), jnp.int32),
                pltpu.VMEM((CH,), jnp.int32),
                pltpu.VMEM((CH,), jnp.int32),
                pltpu.VMEM((CH,), jnp.float32),
                pltpu.VMEM((CH,), jnp.float32),
                pltpu.VMEM((H,), jnp.float32),
                pltpu.SemaphoreType.DMA((6,)),
            ]),
    )
    out = f(hmfs, src, dst, ea, wcol)
    return out.reshape(H, N).T


def _dec_body(h_ref, ag_ref, wu1_ref, wu2_ref, bu_ref, wd_ref, bd_ref, o_ref):
    upd = lax.dot_general(h_ref[...], wu1_ref[...], (((1,), (1,)), ((), ())),
                          preferred_element_type=jnp.float32)
    upd = upd + lax.dot_general(ag_ref[...], wu2_ref[...],
                                (((1,), (1,)), ((), ())),
                                preferred_element_type=jnp.float32)
    upd = upd + bu_ref[...]
    o = jnp.sum(upd * wd_ref[...], axis=1, keepdims=True)
    o_ref[...] = jax.nn.sigmoid(o + bd_ref[0, 0])


def _decode(h, aggr, Wu1, Wu2, b_upd, W_dec, b_dec):
    blk = 1000
    grid = N // blk
    return pl.pallas_call(
        _dec_body,
        grid=(grid,),
        in_specs=[
            pl.BlockSpec((blk, H), lambda i: (i, 0)),
            pl.BlockSpec((blk, H), lambda i: (i, 0)),
            pl.BlockSpec((H, H), lambda i: (0, 0)),
            pl.BlockSpec((H, H), lambda i: (0, 0)),
            pl.BlockSpec((1, H), lambda i: (0, 0)),
            pl.BlockSpec((1, H), lambda i: (0, 0)),
            pl.BlockSpec((1, 1), lambda i: (0, 0)),
        ],
        out_specs=pl.BlockSpec((blk, 1), lambda i: (i, 0)),
        out_shape=jax.ShapeDtypeStruct((N, 1), jnp.float32),
    )(h, aggr, Wu1, Wu2, b_upd.reshape(1, H), W_dec, b_dec.reshape(1, 1))


def kernel(x, edge_index, edge_attr, W_enc, b_enc, W_msg, b_msg,
           W_upd, b_upd, W_dec, b_dec):
    src = edge_index[0]
    dst = edge_index[1]
    Wm1 = W_msg[:, :H]
    wcol = W_msg[:, H]
    Wu1 = W_upd[:, :H]
    Wu2 = W_upd[:, H:]

    h, hm = _encode(x, W_enc, b_enc, Wm1, b_msg)
    hmfs = hm.T.reshape(H * N)
    aggr = _segment_min(hmfs, src, dst, edge_attr, wcol)
    return _decode(h, aggr, Wu1, Wu2, b_upd, W_dec, b_dec)
